# baseline (speedup 1.0000x reference)
import jax
import jax.numpy as jnp
from jax import lax
from jax.experimental import pallas as pl
from jax.experimental.pallas import tpu as pltpu
from jax.experimental.pallas import tpu_sc as plsc

PATH_LEN = 20
EMBED_SIZE = 64
LANES = 16
BLK = 128


def _sc_body(ce_hbm, idx_hbm, bm_hbm, matT_hbm, fac_hbm, res_hbm,
             idx_v, ce_v, bm_v, blk_v, vbuf_v, all_v, out_v, sem0, sem1):
    t = lax.axis_index("s")

    pltpu.sync_copy(idx_hbm, idx_v.at[pl.ds(0, PATH_LEN)])
    pltpu.sync_copy(ce_hbm, ce_v)
    pltpu.sync_copy(bm_hbm, bm_v.at[pl.ds(0, PATH_LEN)])

    lane = lax.iota(jnp.int32, LANES)
    tv = jnp.full((LANES,), t, jnp.int32)
    qv = jnp.where(tv < (PATH_LEN - LANES), tv + LANES, tv)

    def splat(ref, pvec):
        return plsc.load_gather(ref, [pvec])

    def copy_of(pvec, slot, sem):
        s = splat(idx_v, pvec)[0]
        base = pl.multiple_of(s - (s & jnp.int32(BLK - 1)), BLK)
        return pltpu.make_async_copy(
            matT_hbm.at[:, pl.ds(base, BLK)], blk_v.at[slot], sem
        )

    copy_of(tv, 0, sem0).start()
    copy_of(qv, 1, sem1).start()

    cev = [ce_v[pl.ds(c * LANES, LANES)] for c in range(EMBED_SIZE // LANES)]

    def dot_of(slot, ccv):
        acc = plsc.load_gather(blk_v.at[slot], [lane, ccv]) * cev[0]
        for c in range(1, EMBED_SIZE // LANES):
            rows = lane + (c * LANES)
            acc = acc + plsc.load_gather(blk_v.at[slot], [rows, ccv]) * cev[c]
        return jnp.full((LANES,), jnp.sum(acc), jnp.float32)

    copy_of(tv, 0, sem0).wait()
    copy_of(qv, 1, sem1).wait()

    z0 = dot_of(0, splat(idx_v, tv) & jnp.int32(BLK - 1)) * splat(bm_v, tv)
    pr0 = 1.0 / (1.0 + jnp.exp(-z0))

    z1 = dot_of(1, splat(idx_v, qv) & jnp.int32(BLK - 1)) * splat(bm_v, qv)
    pr1 = 1.0 / (1.0 + jnp.exp(-z1))
    pr1 = jnp.where(tv < (PATH_LEN - LANES), pr1, jnp.float32(1.0))

    vbuf_v[...] = pr0 * pr1
    pltpu.sync_copy(vbuf_v, fac_hbm.at[t])
    plsc.subcore_barrier()

    @pl.when(t == 0)
    def _():
        pltpu.sync_copy(fac_hbm, all_v)
        col = plsc.load_gather(all_v, [lane, jnp.zeros((LANES,), jnp.int32)])
        r = col[0]
        for l in range(1, LANES):
            r = r * col[l]
        out_v[...] = jnp.full((LANES,), r, jnp.float32)
        pltpu.sync_copy(out_v, res_hbm)


@jax.jit
def _run(ce, idx, bm, matT):
    mesh = plsc.VectorSubcoreMesh(
        core_axis_name="c", subcore_axis_name="s", num_cores=1
    )
    f = pl.kernel(
        _sc_body,
        out_type=(
            jax.ShapeDtypeStruct((LANES, LANES), jnp.float32),
            jax.ShapeDtypeStruct((LANES,), jnp.float32),
        ),
        mesh=mesh,
        compiler_params=pltpu.CompilerParams(
            needs_layout_passes=False, skip_device_barrier=True
        ),
        scratch_types=[
            pltpu.VMEM((2 * LANES,), jnp.int32),
            pltpu.VMEM((EMBED_SIZE,), jnp.float32),
            pltpu.VMEM((2 * LANES,), jnp.float32),
            pltpu.VMEM((2, EMBED_SIZE, BLK), jnp.float32),
            pltpu.VMEM((LANES,), jnp.float32),
            pltpu.VMEM((LANES, LANES), jnp.float32),
            pltpu.VMEM((LANES,), jnp.float32),
            pltpu.SemaphoreType.DMA,
            pltpu.SemaphoreType.DMA,
        ],
    )
    _, res = f(ce, idx, bm, matT)
    return res[0]


def kernel(context_embedding, input_path_idxs, binary_multiplier, matrix):
    ce = context_embedding.reshape(EMBED_SIZE)
    idx = input_path_idxs.astype(jnp.int32)
    bm = binary_multiplier.reshape(PATH_LEN)
    return _run(ce, idx, bm, matrix.T)


# speedup vs baseline: 1.0012x; 1.0012x over previous
import jax
import jax.numpy as jnp
from jax import lax
from jax.experimental import pallas as pl
from jax.experimental.pallas import tpu as pltpu
from jax.experimental.pallas import tpu_sc as plsc

PATH_LEN = 20
EMBED_SIZE = 64
LANES = 16
BLK = 128


def _sc_body(ce_hbm, idx_hbm, bm_hbm, matT_hbm, fac_hbm, res_hbm,
             idx_v, ce_v, bm_v, blk_v, vbuf_v, all_v, out_v, sem0, sem1):
    t = lax.axis_index("s")

    pltpu.sync_copy(idx_hbm, idx_v.at[pl.ds(0, PATH_LEN)])
    pltpu.sync_copy(ce_hbm, ce_v)
    pltpu.sync_copy(bm_hbm, bm_v.at[pl.ds(0, PATH_LEN)])

    lane = lax.iota(jnp.int32, LANES)
    tv = jnp.full((LANES,), t, jnp.int32)
    qv = jnp.where(tv < (PATH_LEN - LANES), tv + LANES, tv)

    def splat(ref, pvec):
        return plsc.load_gather(ref, [pvec])

    def copy_of(pvec, slot, sem):
        s = splat(idx_v, pvec)[0]
        base = pl.multiple_of(s - (s & jnp.int32(BLK - 1)), BLK)
        return pltpu.make_async_copy(
            matT_hbm.at[:, pl.ds(base, BLK)], blk_v.at[slot], sem
        )

    copy_of(tv, 0, sem0).start()
    copy_of(qv, 1, sem1).start()

    cev = [ce_v[pl.ds(c * LANES, LANES)] for c in range(EMBED_SIZE // LANES)]

    def dot_of(slot, ccv):
        acc = plsc.load_gather(blk_v.at[slot], [lane, ccv]) * cev[0]
        for c in range(1, EMBED_SIZE // LANES):
            rows = lane + (c * LANES)
            acc = acc + plsc.load_gather(blk_v.at[slot], [rows, ccv]) * cev[c]
        return jnp.full((LANES,), jnp.sum(acc), jnp.float32)

    copy_of(tv, 0, sem0).wait()
    copy_of(qv, 1, sem1).wait()

    z0 = dot_of(0, splat(idx_v, tv) & jnp.int32(BLK - 1)) * splat(bm_v, tv)
    pr0 = 1.0 / (1.0 + jnp.exp(-z0))

    z1 = dot_of(1, splat(idx_v, qv) & jnp.int32(BLK - 1)) * splat(bm_v, qv)
    pr1 = 1.0 / (1.0 + jnp.exp(-z1))
    pr1 = jnp.where(tv < (PATH_LEN - LANES), pr1, jnp.float32(1.0))

    vbuf_v[...] = pr0 * pr1
    pltpu.sync_copy(vbuf_v, fac_hbm.at[t])
    plsc.subcore_barrier()

    @pl.when(t == 0)
    def _():
        pltpu.sync_copy(fac_hbm, all_v)
        col = plsc.load_gather(all_v, [lane, jnp.zeros((LANES,), jnp.int32)])
        r = col[0]
        for l in range(1, LANES):
            r = r * col[l]
        out_v[...] = jnp.full((LANES,), r, jnp.float32)
        pltpu.sync_copy(out_v, res_hbm)


@jax.jit
def _run(ce, idx, bm, matT):
    mesh = plsc.VectorSubcoreMesh(
        core_axis_name="c", subcore_axis_name="s", num_cores=1
    )
    f = pl.kernel(
        _sc_body,
        out_type=(
            jax.ShapeDtypeStruct((LANES, LANES), jnp.float32),
            jax.ShapeDtypeStruct((LANES,), jnp.float32),
        ),
        mesh=mesh,
        compiler_params=pltpu.CompilerParams(needs_layout_passes=False),
        scratch_types=[
            pltpu.VMEM((2 * LANES,), jnp.int32),
            pltpu.VMEM((EMBED_SIZE,), jnp.float32),
            pltpu.VMEM((2 * LANES,), jnp.float32),
            pltpu.VMEM((2, EMBED_SIZE, BLK), jnp.float32),
            pltpu.VMEM((LANES,), jnp.float32),
            pltpu.VMEM((LANES, LANES), jnp.float32),
            pltpu.VMEM((LANES,), jnp.float32),
            pltpu.SemaphoreType.DMA,
            pltpu.SemaphoreType.DMA,
        ],
    )
    _, res = f(ce, idx, bm, matT)
    return res[0]


def kernel(context_embedding, input_path_idxs, binary_multiplier, matrix):
    ce = context_embedding.reshape(EMBED_SIZE)
    idx = input_path_idxs.astype(jnp.int32)
    bm = binary_multiplier.reshape(PATH_LEN)
    return _run(ce, idx, bm, matrix.T)
